# baseline (speedup 1.0000x reference)
import functools

import jax
import jax.numpy as jnp
from jax import lax
from jax.experimental import pallas as pl
from jax.experimental.pallas import tpu as pltpu
from jax.experimental.pallas import tpu_sc as plsc

N = 10000
E = 320000
D = 128
H1 = 512
H2 = 256

CHUNK = 128
GRP = 2
NGRP = E // (CHUNK * GRP)
NTILES = 16
WRT = 624
WTAIL = N - NTILES * WRT


def _sc_agg_body(x_hbm, edge_sj_hbm, edge_sr_hbm, zeros_hbm, out_hbm,
                 eidx, rows3, acc_sh, isem, gsem, ssem):
    r = lax.axis_index("c")
    s = lax.axis_index("s")
    n_my = (NGRP - 1 - s) // NTILES + 1
    t_total = n_my * GRP

    zstart = s * WRT
    pltpu.sync_copy(zeros_hbm.at[pl.ds(0, WRT)], acc_sh.at[pl.ds(zstart, WRT)])

    @pl.when(s == NTILES - 1)
    def _():
        pltpu.sync_copy(zeros_hbm.at[pl.ds(0, WTAIL)],
                        acc_sh.at[pl.ds(NTILES * WRT, WTAIL)])

    def _run(edge_hbm):
        pltpu.sync_copy(edge_hbm.at[:, s], eidx.at[0])
        pltpu.sync_copy(edge_hbm.at[:, NTILES + s], eidx.at[1])
        plsc.subcore_barrier()

        pltpu.async_copy(x_hbm.at[eidx.at[0, 0, 0]], rows3.at[0], gsem)
        pltpu.async_copy(x_hbm.at[eidx.at[0, 0, 1]], rows3.at[1], gsem)

        def _step(t, carry):
            i, j = lax.div(t, GRP), lax.rem(t, GRP)
            b = lax.rem(t, 3)
            ib = lax.rem(i, 3)
            pltpu.make_async_copy(x_hbm.at[pl.ds(0, CHUNK)],
                                  rows3.at[b], gsem).wait()
            pltpu.async_copy(rows3.at[b], acc_sh.at[eidx.at[ib, 1, j]],
                             ssem, add=True)

            @pl.when(t + 2 < t_total)
            def _():
                @pl.when(t >= 1)
                def _():
                    pltpu.make_async_copy(x_hbm.at[pl.ds(0, CHUNK)],
                                          rows3.at[0], ssem).wait()

                @pl.when(jnp.logical_and(j == 0, i + 2 < n_my))
                def _():
                    g = (i + 2) * NTILES + s
                    pltpu.async_copy(edge_hbm.at[:, g],
                                     eidx.at[lax.rem(i + 2, 3)], isem)

                tn = t + 2
                i2, j2 = lax.div(tn, GRP), lax.rem(tn, GRP)

                @pl.when(jnp.logical_and(j2 == 0, i2 >= 2))
                def _():
                    pltpu.make_async_copy(edge_hbm.at[:, s], eidx.at[0],
                                          isem).wait()

                pltpu.async_copy(x_hbm.at[eidx.at[lax.rem(i2, 3), 0, j2]],
                                 rows3.at[lax.rem(tn, 3)], gsem)

            return carry

        lax.fori_loop(0, t_total, _step, 0)
        for _ in range(3):
            pltpu.make_async_copy(x_hbm.at[pl.ds(0, CHUNK)],
                                  rows3.at[0], ssem).wait()

    @pl.when(r == 0)
    def _():
        _run(edge_sj_hbm)

    @pl.when(r == 1)
    def _():
        _run(edge_sr_hbm)

    plsc.subcore_barrier()

    pltpu.sync_copy(acc_sh.at[pl.ds(zstart, WRT)],
                    out_hbm.at[r, pl.ds(zstart, WRT)])

    @pl.when(s == NTILES - 1)
    def _():
        pltpu.sync_copy(acc_sh.at[pl.ds(NTILES * WRT, WTAIL)],
                        out_hbm.at[r, pl.ds(NTILES * WRT, WTAIL)])


_sc_agg = functools.partial(
    pl.kernel,
    out_type=jax.ShapeDtypeStruct((2, N, D), jnp.float32),
    mesh=plsc.VectorSubcoreMesh(core_axis_name="c", subcore_axis_name="s"),
    scratch_types=[
        pltpu.VMEM((3, 2, GRP, CHUNK), jnp.int32),
        pltpu.VMEM((3, CHUNK, D), jnp.float32),
        pltpu.VMEM_SHARED((N, D), jnp.float32),
        pltpu.SemaphoreType.DMA,
        pltpu.SemaphoreType.DMA,
        pltpu.SemaphoreType.DMA,
    ],
)(_sc_agg_body)


RB = 1000


def _tc_root_body(xj_ref, xr_ref, wrootj_ref, wrootr_ref, bj_ref, br_ref,
                  rootj_ref, rootr_ref):
    f32 = jnp.float32
    rootj_ref[...] = (jnp.dot(xj_ref[...], wrootj_ref[...],
                              preferred_element_type=f32) + bj_ref[...])
    rootr_ref[...] = (jnp.dot(xr_ref[...], wrootr_ref[...],
                              preferred_element_type=f32) + br_ref[...])


def _tc_mlp_body(aggj_ref, rootj_ref, aggr_ref, rootr_ref,
                 wrelj_ref, wrelr_ref,
                 wm1a_ref, wm1b_ref, bm1_ref,
                 wm2_ref, bm2_ref, wm3_ref, bm3_ref, out_ref):
    f32 = jnp.float32
    hj = (jnp.dot(aggj_ref[...], wrelj_ref[...], preferred_element_type=f32)
          + rootj_ref[...])
    hj = jnp.maximum(hj, 0.0)
    hr = (jnp.dot(aggr_ref[...], wrelr_ref[...], preferred_element_type=f32)
          + rootr_ref[...])
    hr = jnp.maximum(hr, 0.0)
    h1 = (jnp.dot(hj, wm1a_ref[...], preferred_element_type=f32)
          + jnp.dot(hr, wm1b_ref[...], preferred_element_type=f32)
          + bm1_ref[...])
    h1 = jnp.maximum(h1, 0.0)
    h2 = jnp.maximum(
        jnp.dot(h1, wm2_ref[...], preferred_element_type=f32) + bm2_ref[...],
        0.0)
    out_ref[...] = (jnp.sum(h2 * wm3_ref[...], axis=1, keepdims=True)
                    + bm3_ref[...])


def _full_spec(shape):
    return pl.BlockSpec(shape, lambda i: (0,) * len(shape))


def _row_spec():
    return pl.BlockSpec((RB, D), lambda i: (i, 0))


_tc_root = pl.pallas_call(
    _tc_root_body,
    grid=(N // RB,),
    in_specs=[
        _row_spec(), _row_spec(),
        _full_spec((D, D)), _full_spec((D, D)),
        _full_spec((1, D)), _full_spec((1, D)),
    ],
    out_specs=[_row_spec(), _row_spec()],
    out_shape=[jax.ShapeDtypeStruct((N, D), jnp.float32),
               jax.ShapeDtypeStruct((N, D), jnp.float32)],
)

_tc_mlp = pl.pallas_call(
    _tc_mlp_body,
    grid=(N // RB,),
    in_specs=[
        _row_spec(), _row_spec(), _row_spec(), _row_spec(),
        _full_spec((D, D)), _full_spec((D, D)),
        _full_spec((D, H1)), _full_spec((D, H1)), _full_spec((1, H1)),
        _full_spec((H1, H2)), _full_spec((1, H2)),
        _full_spec((1, H2)), _full_spec((1, 1)),
    ],
    out_specs=pl.BlockSpec((RB, 1), lambda i: (i, 0)),
    out_shape=jax.ShapeDtypeStruct((N, 1), jnp.float32),
)


def kernel(x_skill, x_job, x_resume, edge_index_skill_job,
           edge_index_skill_resume, W_rel_sj, b_rel_sj, W_root_sj, W_rel_sr,
           b_rel_sr, W_root_sr, Wm1, bm1, Wm2, bm2, Wm3, bm3):
    edge_sj = edge_index_skill_job.reshape(2, NGRP, GRP, CHUNK)
    edge_sr = edge_index_skill_resume.reshape(2, NGRP, GRP, CHUNK)
    zeros = jnp.zeros((WRT, D), jnp.float32)
    agg = _sc_agg(x_skill, edge_sj, edge_sr, zeros)

    rootj, rootr = _tc_root(
        x_job, x_resume, W_root_sj.T, W_root_sr.T,
        b_rel_sj.reshape(1, D), b_rel_sr.reshape(1, D),
    )
    out = _tc_mlp(
        agg[0], rootj, agg[1], rootr,
        W_rel_sj.T, W_rel_sr.T,
        Wm1.T[:D], Wm1.T[D:], bm1.reshape(1, H1),
        Wm2.T, bm2.reshape(1, H2),
        Wm3, bm3.reshape(1, 1),
    )
    return out.reshape(N)


# speedup vs baseline: 1.0040x; 1.0040x over previous
import functools

import jax
import jax.numpy as jnp
from jax import lax
from jax.experimental import pallas as pl
from jax.experimental.pallas import tpu as pltpu
from jax.experimental.pallas import tpu_sc as plsc

N = 10000
E = 320000
D = 128
H1 = 512
H2 = 256

CHUNK = 128
GRP = 2
NGRP = E // (CHUNK * GRP)
NTILES = 16
WRT = 624
WTAIL = N - NTILES * WRT


def _sc_agg_body(x_hbm, edge_sj_hbm, edge_sr_hbm, zeros_hbm, out_hbm,
                 eidx, rows3, acc_sh, isem, gsem, ssem):
    r = lax.axis_index("c")
    s = lax.axis_index("s")
    n_my = (NGRP - 1 - s) // NTILES + 1
    t_total = n_my * GRP

    zstart = s * WRT
    pltpu.sync_copy(zeros_hbm.at[pl.ds(0, WRT)], acc_sh.at[pl.ds(zstart, WRT)])

    @pl.when(s == NTILES - 1)
    def _():
        pltpu.sync_copy(zeros_hbm.at[pl.ds(0, WTAIL)],
                        acc_sh.at[pl.ds(NTILES * WRT, WTAIL)])

    def _run(edge_hbm):
        pltpu.sync_copy(edge_hbm.at[:, s], eidx.at[0])
        pltpu.sync_copy(edge_hbm.at[:, NTILES + s], eidx.at[1])
        plsc.subcore_barrier()

        pltpu.async_copy(x_hbm.at[eidx.at[0, 0, 0]], rows3.at[0], gsem)
        pltpu.async_copy(x_hbm.at[eidx.at[0, 0, 1]], rows3.at[1], gsem)

        def _step(t, carry):
            i, j = lax.div(t, GRP), lax.rem(t, GRP)
            b = lax.rem(t, 3)
            ib = lax.rem(i, 3)
            pltpu.make_async_copy(x_hbm.at[pl.ds(0, CHUNK)],
                                  rows3.at[b], gsem).wait()
            pltpu.async_copy(rows3.at[b], acc_sh.at[eidx.at[ib, 1, j]],
                             ssem, add=True)

            @pl.when(t + 2 < t_total)
            def _():
                @pl.when(t >= 1)
                def _():
                    pltpu.make_async_copy(x_hbm.at[pl.ds(0, CHUNK)],
                                          rows3.at[0], ssem).wait()

                @pl.when(jnp.logical_and(j == 0, i + 2 < n_my))
                def _():
                    g = (i + 2) * NTILES + s
                    pltpu.async_copy(edge_hbm.at[:, g],
                                     eidx.at[lax.rem(i + 2, 3)], isem)

                tn = t + 2
                i2, j2 = lax.div(tn, GRP), lax.rem(tn, GRP)

                @pl.when(jnp.logical_and(j2 == 0, i2 >= 2))
                def _():
                    pltpu.make_async_copy(edge_hbm.at[:, s], eidx.at[0],
                                          isem).wait()

                pltpu.async_copy(x_hbm.at[eidx.at[lax.rem(i2, 3), 0, j2]],
                                 rows3.at[lax.rem(tn, 3)], gsem)

            return carry

        lax.fori_loop(0, t_total, _step, 0)
        for _ in range(3):
            pltpu.make_async_copy(x_hbm.at[pl.ds(0, CHUNK)],
                                  rows3.at[0], ssem).wait()

    @pl.when(r == 0)
    def _():
        _run(edge_sj_hbm)

    @pl.when(r == 1)
    def _():
        _run(edge_sr_hbm)

    plsc.subcore_barrier()

    pltpu.sync_copy(acc_sh.at[pl.ds(zstart, WRT)],
                    out_hbm.at[r, pl.ds(zstart, WRT)])

    @pl.when(s == NTILES - 1)
    def _():
        pltpu.sync_copy(acc_sh.at[pl.ds(NTILES * WRT, WTAIL)],
                        out_hbm.at[r, pl.ds(NTILES * WRT, WTAIL)])


_sc_agg = functools.partial(
    pl.kernel,
    out_type=jax.ShapeDtypeStruct((2, N, D), jnp.float32),
    mesh=plsc.VectorSubcoreMesh(core_axis_name="c", subcore_axis_name="s"),
    scratch_types=[
        pltpu.VMEM((3, 2, GRP, CHUNK), jnp.int32),
        pltpu.VMEM((3, CHUNK, D), jnp.float32),
        pltpu.VMEM_SHARED((N, D), jnp.float32),
        pltpu.SemaphoreType.DMA,
        pltpu.SemaphoreType.DMA,
        pltpu.SemaphoreType.DMA,
    ],
)(_sc_agg_body)


RB = 1000


def _tc_root_body(xj_ref, xr_ref, wrootj_ref, wrootr_ref, bj_ref, br_ref,
                  rootj_ref, rootr_ref):
    f32 = jnp.float32
    rootj_ref[...] = (jnp.dot(xj_ref[...], wrootj_ref[...],
                              preferred_element_type=f32) + bj_ref[...])
    rootr_ref[...] = (jnp.dot(xr_ref[...], wrootr_ref[...],
                              preferred_element_type=f32) + br_ref[...])


def _tc_mlp_body(aggj_ref, rootj_ref, aggr_ref, rootr_ref,
                 wrelj_ref, wrelr_ref,
                 wm1a_ref, wm1b_ref, bm1_ref,
                 wm2_ref, bm2_ref, wm3_ref, bm3_ref, out_ref):
    f32 = jnp.float32
    hj = (jnp.dot(aggj_ref[...], wrelj_ref[...], preferred_element_type=f32)
          + rootj_ref[...])
    hj = jnp.maximum(hj, 0.0)
    hr = (jnp.dot(aggr_ref[...], wrelr_ref[...], preferred_element_type=f32)
          + rootr_ref[...])
    hr = jnp.maximum(hr, 0.0)
    bf16 = jnp.bfloat16
    h1 = (jnp.dot(hj.astype(bf16), wm1a_ref[...], preferred_element_type=f32)
          + jnp.dot(hr.astype(bf16), wm1b_ref[...], preferred_element_type=f32)
          + bm1_ref[...])
    h1 = jnp.maximum(h1, 0.0)
    h2 = jnp.maximum(
        jnp.dot(h1.astype(bf16), wm2_ref[...], preferred_element_type=f32)
        + bm2_ref[...],
        0.0)
    out_ref[...] = (jnp.sum(h2 * wm3_ref[...], axis=1, keepdims=True)
                    + bm3_ref[...])


def _full_spec(shape):
    return pl.BlockSpec(shape, lambda i: (0,) * len(shape))


def _row_spec():
    return pl.BlockSpec((RB, D), lambda i: (i, 0))


_tc_root = pl.pallas_call(
    _tc_root_body,
    grid=(N // RB,),
    in_specs=[
        _row_spec(), _row_spec(),
        _full_spec((D, D)), _full_spec((D, D)),
        _full_spec((1, D)), _full_spec((1, D)),
    ],
    out_specs=[_row_spec(), _row_spec()],
    out_shape=[jax.ShapeDtypeStruct((N, D), jnp.float32),
               jax.ShapeDtypeStruct((N, D), jnp.float32)],
)

_tc_mlp = pl.pallas_call(
    _tc_mlp_body,
    grid=(N // RB,),
    in_specs=[
        _row_spec(), _row_spec(), _row_spec(), _row_spec(),
        _full_spec((D, D)), _full_spec((D, D)),
        _full_spec((D, H1)), _full_spec((D, H1)), _full_spec((1, H1)),
        _full_spec((H1, H2)), _full_spec((1, H2)),
        _full_spec((1, H2)), _full_spec((1, 1)),
    ],
    out_specs=pl.BlockSpec((RB, 1), lambda i: (i, 0)),
    out_shape=jax.ShapeDtypeStruct((N, 1), jnp.float32),
)


def kernel(x_skill, x_job, x_resume, edge_index_skill_job,
           edge_index_skill_resume, W_rel_sj, b_rel_sj, W_root_sj, W_rel_sr,
           b_rel_sr, W_root_sr, Wm1, bm1, Wm2, bm2, Wm3, bm3):
    edge_sj = edge_index_skill_job.reshape(2, NGRP, GRP, CHUNK)
    edge_sr = edge_index_skill_resume.reshape(2, NGRP, GRP, CHUNK)
    zeros = jnp.zeros((WRT, D), jnp.float32)
    agg = _sc_agg(x_skill, edge_sj, edge_sr, zeros)

    rootj, rootr = _tc_root(
        x_job, x_resume, W_root_sj.T, W_root_sr.T,
        b_rel_sj.reshape(1, D), b_rel_sr.reshape(1, D),
    )
    out = _tc_mlp(
        agg[0], rootj, agg[1], rootr,
        W_rel_sj.T, W_rel_sr.T,
        Wm1.T[:D].astype(jnp.bfloat16), Wm1.T[D:].astype(jnp.bfloat16),
        bm1.reshape(1, H1),
        Wm2.T.astype(jnp.bfloat16), bm2.reshape(1, H2),
        Wm3, bm3.reshape(1, 1),
    )
    return out.reshape(N)


# speedup vs baseline: 1.0049x; 1.0009x over previous
import functools

import jax
import jax.numpy as jnp
from jax import lax
from jax.experimental import pallas as pl
from jax.experimental.pallas import tpu as pltpu
from jax.experimental.pallas import tpu_sc as plsc

N = 10000
E = 320000
D = 128
H1 = 512
H2 = 256

CHUNK = 80
GRP = 5
NGRP = E // (CHUNK * GRP)
NTILES = 16
WRT = 624
WTAIL = N - NTILES * WRT


def _sc_agg_body(x_hbm, edge_sj_hbm, edge_sr_hbm, zeros_hbm, out_hbm,
                 eidx, rows3, acc_sh, isem, gsem, ssem):
    r = lax.axis_index("c")
    s = lax.axis_index("s")
    n_my = (NGRP - 1 - s) // NTILES + 1
    t_total = n_my * GRP

    zstart = s * WRT
    pltpu.sync_copy(zeros_hbm.at[pl.ds(0, WRT)], acc_sh.at[pl.ds(zstart, WRT)])

    @pl.when(s == NTILES - 1)
    def _():
        pltpu.sync_copy(zeros_hbm.at[pl.ds(0, WTAIL)],
                        acc_sh.at[pl.ds(NTILES * WRT, WTAIL)])

    def _run(edge_hbm):
        pltpu.sync_copy(edge_hbm.at[:, s], eidx.at[0])
        pltpu.sync_copy(edge_hbm.at[:, NTILES + s], eidx.at[1])
        plsc.subcore_barrier()

        pltpu.async_copy(x_hbm.at[eidx.at[0, 0, 0]], rows3.at[0], gsem)
        pltpu.async_copy(x_hbm.at[eidx.at[0, 0, 1]], rows3.at[1], gsem)
        pltpu.async_copy(x_hbm.at[eidx.at[0, 0, 2]], rows3.at[2], gsem)

        def _step(t, carry):
            i, j = lax.div(t, GRP), lax.rem(t, GRP)
            b = lax.rem(t, 4)
            ib = lax.rem(i, 3)
            pltpu.make_async_copy(x_hbm.at[pl.ds(0, CHUNK)],
                                  rows3.at[b], gsem).wait()
            pltpu.async_copy(rows3.at[b], acc_sh.at[eidx.at[ib, 1, j]],
                             ssem, add=True)

            @pl.when(t + 3 < t_total)
            def _():
                @pl.when(t >= 1)
                def _():
                    pltpu.make_async_copy(x_hbm.at[pl.ds(0, CHUNK)],
                                          rows3.at[0], ssem).wait()

                @pl.when(jnp.logical_and(j == 0, i + 2 < n_my))
                def _():
                    g = (i + 2) * NTILES + s
                    pltpu.async_copy(edge_hbm.at[:, g],
                                     eidx.at[lax.rem(i + 2, 3)], isem)

                tn = t + 3
                i2, j2 = lax.div(tn, GRP), lax.rem(tn, GRP)

                @pl.when(jnp.logical_and(j2 == 0, i2 >= 2))
                def _():
                    pltpu.make_async_copy(edge_hbm.at[:, s], eidx.at[0],
                                          isem).wait()

                pltpu.async_copy(x_hbm.at[eidx.at[lax.rem(i2, 3), 0, j2]],
                                 rows3.at[lax.rem(tn, 4)], gsem)

            return carry

        lax.fori_loop(0, t_total, _step, 0)
        for _ in range(4):
            pltpu.make_async_copy(x_hbm.at[pl.ds(0, CHUNK)],
                                  rows3.at[0], ssem).wait()

    @pl.when(r == 0)
    def _():
        _run(edge_sj_hbm)

    @pl.when(r == 1)
    def _():
        _run(edge_sr_hbm)

    plsc.subcore_barrier()

    pltpu.sync_copy(acc_sh.at[pl.ds(zstart, WRT)],
                    out_hbm.at[r, pl.ds(zstart, WRT)])

    @pl.when(s == NTILES - 1)
    def _():
        pltpu.sync_copy(acc_sh.at[pl.ds(NTILES * WRT, WTAIL)],
                        out_hbm.at[r, pl.ds(NTILES * WRT, WTAIL)])


_sc_agg = functools.partial(
    pl.kernel,
    out_type=jax.ShapeDtypeStruct((2, N, D), jnp.float32),
    mesh=plsc.VectorSubcoreMesh(core_axis_name="c", subcore_axis_name="s"),
    scratch_types=[
        pltpu.VMEM((3, 2, GRP, CHUNK), jnp.int32),
        pltpu.VMEM((4, CHUNK, D), jnp.float32),
        pltpu.VMEM_SHARED((N, D), jnp.float32),
        pltpu.SemaphoreType.DMA,
        pltpu.SemaphoreType.DMA,
        pltpu.SemaphoreType.DMA,
    ],
)(_sc_agg_body)


RB = 1000


def _tc_root_body(xj_ref, xr_ref, wrootj_ref, wrootr_ref, bj_ref, br_ref,
                  rootj_ref, rootr_ref):
    f32 = jnp.float32
    rootj_ref[...] = (jnp.dot(xj_ref[...], wrootj_ref[...],
                              preferred_element_type=f32) + bj_ref[...])
    rootr_ref[...] = (jnp.dot(xr_ref[...], wrootr_ref[...],
                              preferred_element_type=f32) + br_ref[...])


def _tc_mlp_body(aggj_ref, rootj_ref, aggr_ref, rootr_ref,
                 wrelj_ref, wrelr_ref,
                 wm1a_ref, wm1b_ref, bm1_ref,
                 wm2_ref, bm2_ref, wm3_ref, bm3_ref, out_ref):
    f32 = jnp.float32
    hj = (jnp.dot(aggj_ref[...], wrelj_ref[...], preferred_element_type=f32)
          + rootj_ref[...])
    hj = jnp.maximum(hj, 0.0)
    hr = (jnp.dot(aggr_ref[...], wrelr_ref[...], preferred_element_type=f32)
          + rootr_ref[...])
    hr = jnp.maximum(hr, 0.0)
    bf16 = jnp.bfloat16
    h1 = (jnp.dot(hj.astype(bf16), wm1a_ref[...], preferred_element_type=f32)
          + jnp.dot(hr.astype(bf16), wm1b_ref[...], preferred_element_type=f32)
          + bm1_ref[...])
    h1 = jnp.maximum(h1, 0.0)
    h2 = jnp.maximum(
        jnp.dot(h1.astype(bf16), wm2_ref[...], preferred_element_type=f32)
        + bm2_ref[...],
        0.0)
    out_ref[...] = (jnp.sum(h2 * wm3_ref[...], axis=1, keepdims=True)
                    + bm3_ref[...])


def _full_spec(shape):
    return pl.BlockSpec(shape, lambda i: (0,) * len(shape))


def _row_spec():
    return pl.BlockSpec((RB, D), lambda i: (i, 0))


_tc_root = pl.pallas_call(
    _tc_root_body,
    grid=(N // RB,),
    in_specs=[
        _row_spec(), _row_spec(),
        _full_spec((D, D)), _full_spec((D, D)),
        _full_spec((1, D)), _full_spec((1, D)),
    ],
    out_specs=[_row_spec(), _row_spec()],
    out_shape=[jax.ShapeDtypeStruct((N, D), jnp.float32),
               jax.ShapeDtypeStruct((N, D), jnp.float32)],
)

_tc_mlp = pl.pallas_call(
    _tc_mlp_body,
    grid=(N // RB,),
    in_specs=[
        _row_spec(), _row_spec(), _row_spec(), _row_spec(),
        _full_spec((D, D)), _full_spec((D, D)),
        _full_spec((D, H1)), _full_spec((D, H1)), _full_spec((1, H1)),
        _full_spec((H1, H2)), _full_spec((1, H2)),
        _full_spec((1, H2)), _full_spec((1, 1)),
    ],
    out_specs=pl.BlockSpec((RB, 1), lambda i: (i, 0)),
    out_shape=jax.ShapeDtypeStruct((N, 1), jnp.float32),
)


def kernel(x_skill, x_job, x_resume, edge_index_skill_job,
           edge_index_skill_resume, W_rel_sj, b_rel_sj, W_root_sj, W_rel_sr,
           b_rel_sr, W_root_sr, Wm1, bm1, Wm2, bm2, Wm3, bm3):
    edge_sj = edge_index_skill_job.reshape(2, NGRP, GRP, CHUNK)
    edge_sr = edge_index_skill_resume.reshape(2, NGRP, GRP, CHUNK)
    zeros = jnp.zeros((WRT, D), jnp.float32)
    agg = _sc_agg(x_skill, edge_sj, edge_sr, zeros)

    rootj, rootr = _tc_root(
        x_job, x_resume, W_root_sj.T, W_root_sr.T,
        b_rel_sj.reshape(1, D), b_rel_sr.reshape(1, D),
    )
    out = _tc_mlp(
        agg[0], rootj, agg[1], rootr,
        W_rel_sj.T, W_rel_sr.T,
        Wm1.T[:D].astype(jnp.bfloat16), Wm1.T[D:].astype(jnp.bfloat16),
        bm1.reshape(1, H1),
        Wm2.T.astype(jnp.bfloat16), bm2.reshape(1, H2),
        Wm3, bm3.reshape(1, 1),
    )
    return out.reshape(N)


# speedup vs baseline: 1.0134x; 1.0085x over previous
import functools

import jax
import jax.numpy as jnp
from jax import lax
from jax.experimental import pallas as pl
from jax.experimental.pallas import tpu as pltpu
from jax.experimental.pallas import tpu_sc as plsc

N = 10000
E = 320000
D = 128
H1 = 512
H2 = 256

CHUNK = 80
GRP = 5
NGRP = E // (CHUNK * GRP)
NTILES = 16
WRT = 624
WTAIL = N - NTILES * WRT


def _sc_agg_body(x_hbm, edge_sj_hbm, edge_sr_hbm, zeros_hbm, out_hbm,
                 eidx, rows3, acc_sh, isem, gsem, ssem):
    r = lax.axis_index("c")
    s = lax.axis_index("s")
    n_my = (NGRP - 1 - s) // NTILES + 1
    t_total = n_my * GRP

    zstart = s * WRT
    pltpu.sync_copy(zeros_hbm.at[pl.ds(0, WRT)], acc_sh.at[pl.ds(zstart, WRT)])

    @pl.when(s == NTILES - 1)
    def _():
        pltpu.sync_copy(zeros_hbm.at[pl.ds(0, WTAIL)],
                        acc_sh.at[pl.ds(NTILES * WRT, WTAIL)])

    def _run(edge_hbm):
        pltpu.sync_copy(edge_hbm.at[:, s], eidx.at[0])
        pltpu.sync_copy(edge_hbm.at[:, NTILES + s], eidx.at[1])
        plsc.subcore_barrier()

        pltpu.async_copy(x_hbm.at[eidx.at[0, 0, 0]], rows3.at[0], gsem)
        pltpu.async_copy(x_hbm.at[eidx.at[0, 0, 1]], rows3.at[1], gsem)
        pltpu.async_copy(x_hbm.at[eidx.at[0, 0, 2]], rows3.at[2], gsem)

        def _step(t, carry):
            i, j = lax.div(t, GRP), lax.rem(t, GRP)
            b = lax.rem(t, 4)
            ib = lax.rem(i, 3)
            pltpu.make_async_copy(x_hbm.at[pl.ds(0, CHUNK)],
                                  rows3.at[b], gsem).wait()
            pltpu.async_copy(rows3.at[b], acc_sh.at[eidx.at[ib, 1, j]],
                             ssem, add=True)

            @pl.when(t + 3 < t_total)
            def _():
                @pl.when(t >= 1)
                def _():
                    pltpu.make_async_copy(x_hbm.at[pl.ds(0, CHUNK)],
                                          rows3.at[0], ssem).wait()

                @pl.when(jnp.logical_and(j == 0, i + 2 < n_my))
                def _():
                    g = (i + 2) * NTILES + s
                    pltpu.async_copy(edge_hbm.at[:, g],
                                     eidx.at[lax.rem(i + 2, 3)], isem)

                tn = t + 3
                i2, j2 = lax.div(tn, GRP), lax.rem(tn, GRP)

                @pl.when(jnp.logical_and(j2 == 0, i2 >= 2))
                def _():
                    pltpu.make_async_copy(edge_hbm.at[:, s], eidx.at[0],
                                          isem).wait()

                pltpu.async_copy(x_hbm.at[eidx.at[lax.rem(i2, 3), 0, j2]],
                                 rows3.at[lax.rem(tn, 4)], gsem)

            return carry

        lax.fori_loop(0, t_total, _step, 0)
        for _ in range(4):
            pltpu.make_async_copy(x_hbm.at[pl.ds(0, CHUNK)],
                                  rows3.at[0], ssem).wait()

    @pl.when(r == 0)
    def _():
        _run(edge_sj_hbm)

    @pl.when(r == 1)
    def _():
        _run(edge_sr_hbm)

    plsc.subcore_barrier()

    pltpu.sync_copy(acc_sh.at[pl.ds(zstart, WRT)],
                    out_hbm.at[r, pl.ds(zstart, WRT)])

    @pl.when(s == NTILES - 1)
    def _():
        pltpu.sync_copy(acc_sh.at[pl.ds(NTILES * WRT, WTAIL)],
                        out_hbm.at[r, pl.ds(NTILES * WRT, WTAIL)])


_sc_agg = functools.partial(
    pl.kernel,
    out_type=jax.ShapeDtypeStruct((2, N, D), jnp.float32),
    mesh=plsc.VectorSubcoreMesh(core_axis_name="c", subcore_axis_name="s"),
    scratch_types=[
        pltpu.VMEM((3, 2, GRP, CHUNK), jnp.int32),
        pltpu.VMEM((4, CHUNK, D), jnp.float32),
        pltpu.VMEM_SHARED((N, D), jnp.float32),
        pltpu.SemaphoreType.DMA,
        pltpu.SemaphoreType.DMA,
        pltpu.SemaphoreType.DMA,
    ],
)(_sc_agg_body)


RB = 1000


def _tc_epilogue_body(aggj_ref, xj_ref, aggr_ref, xr_ref,
                      wrelj_ref, wrootj_ref, bj_ref,
                      wrelr_ref, wrootr_ref, br_ref,
                      wm1a_ref, wm1b_ref, bm1_ref,
                      wm2_ref, bm2_ref, wm3_ref, bm3_ref, out_ref):
    f32 = jnp.float32
    bf16 = jnp.bfloat16
    hj = (jnp.dot(aggj_ref[...], wrelj_ref[...], preferred_element_type=f32)
          + jnp.dot(xj_ref[...], wrootj_ref[...], preferred_element_type=f32)
          + bj_ref[...])
    hj = jnp.maximum(hj, 0.0)
    hr = (jnp.dot(aggr_ref[...], wrelr_ref[...], preferred_element_type=f32)
          + jnp.dot(xr_ref[...], wrootr_ref[...], preferred_element_type=f32)
          + br_ref[...])
    hr = jnp.maximum(hr, 0.0)
    h1 = (jnp.dot(hj.astype(bf16), wm1a_ref[...], preferred_element_type=f32)
          + jnp.dot(hr.astype(bf16), wm1b_ref[...], preferred_element_type=f32)
          + bm1_ref[...])
    h1 = jnp.maximum(h1, 0.0)
    h2 = jnp.maximum(
        jnp.dot(h1.astype(bf16), wm2_ref[...], preferred_element_type=f32)
        + bm2_ref[...],
        0.0)
    out_ref[...] = (jnp.sum(h2 * wm3_ref[...], axis=1, keepdims=True)
                    + bm3_ref[...])


def _full_spec(shape):
    return pl.BlockSpec(shape, lambda i: (0,) * len(shape))


def _row_spec():
    return pl.BlockSpec((RB, D), lambda i: (i, 0))


_tc_epilogue = pl.pallas_call(
    _tc_epilogue_body,
    grid=(N // RB,),
    in_specs=[
        _row_spec(), _row_spec(), _row_spec(), _row_spec(),
        _full_spec((D, D)), _full_spec((D, D)), _full_spec((1, D)),
        _full_spec((D, D)), _full_spec((D, D)), _full_spec((1, D)),
        _full_spec((D, H1)), _full_spec((D, H1)), _full_spec((1, H1)),
        _full_spec((H1, H2)), _full_spec((1, H2)),
        _full_spec((1, H2)), _full_spec((1, 1)),
    ],
    out_specs=pl.BlockSpec((RB, 1), lambda i: (i, 0)),
    out_shape=jax.ShapeDtypeStruct((N, 1), jnp.float32),
)


def kernel(x_skill, x_job, x_resume, edge_index_skill_job,
           edge_index_skill_resume, W_rel_sj, b_rel_sj, W_root_sj, W_rel_sr,
           b_rel_sr, W_root_sr, Wm1, bm1, Wm2, bm2, Wm3, bm3):
    edge_sj = edge_index_skill_job.reshape(2, NGRP, GRP, CHUNK)
    edge_sr = edge_index_skill_resume.reshape(2, NGRP, GRP, CHUNK)
    zeros = jnp.zeros((WRT, D), jnp.float32)
    agg = _sc_agg(x_skill, edge_sj, edge_sr, zeros)

    out = _tc_epilogue(
        agg[0], x_job, agg[1], x_resume,
        W_rel_sj.T, W_root_sj.T, b_rel_sj.reshape(1, D),
        W_rel_sr.T, W_root_sr.T, b_rel_sr.reshape(1, D),
        Wm1.T[:D].astype(jnp.bfloat16), Wm1.T[D:].astype(jnp.bfloat16),
        bm1.reshape(1, H1),
        Wm2.T.astype(jnp.bfloat16), bm2.reshape(1, H2),
        Wm3, bm3.reshape(1, 1),
    )
    return out.reshape(N)


# speedup vs baseline: 1.0299x; 1.0163x over previous
import functools

import jax
import jax.numpy as jnp
from jax import lax
from jax.experimental import pallas as pl
from jax.experimental.pallas import tpu as pltpu
from jax.experimental.pallas import tpu_sc as plsc

N = 10000
E = 320000
D = 128
H1 = 512
H2 = 256

CHUNK = 80
GRP = 5
NGRP = E // (CHUNK * GRP)
NTILES = 16
WRT = 624
WTAIL = N - NTILES * WRT


def _sc_agg_body(x_hbm, edge_sj_hbm, edge_sr_hbm, zeros_hbm, out_hbm,
                 eidx, rows3, acc_sh, isem, gsem, ssem):
    r = lax.axis_index("c")
    s = lax.axis_index("s")
    n_my = (NGRP - 1 - s) // NTILES + 1
    t_total = n_my * GRP

    zstart = s * WRT
    pltpu.sync_copy(zeros_hbm.at[pl.ds(0, WRT)], acc_sh.at[pl.ds(zstart, WRT)])

    @pl.when(s == NTILES - 1)
    def _():
        pltpu.sync_copy(zeros_hbm.at[pl.ds(0, WTAIL)],
                        acc_sh.at[pl.ds(NTILES * WRT, WTAIL)])

    def _run(edge_hbm):
        pltpu.sync_copy(edge_hbm.at[:, s], eidx.at[0])
        pltpu.sync_copy(edge_hbm.at[:, NTILES + s], eidx.at[1])
        plsc.subcore_barrier()

        pltpu.async_copy(x_hbm.at[eidx.at[0, 0, 0]], rows3.at[0], gsem)
        pltpu.async_copy(x_hbm.at[eidx.at[0, 0, 1]], rows3.at[1], gsem)
        pltpu.async_copy(x_hbm.at[eidx.at[0, 0, 2]], rows3.at[2], gsem)

        def _step(t, carry):
            i, j = lax.div(t, GRP), lax.rem(t, GRP)
            b = lax.rem(t, 4)
            ib = lax.rem(i, 3)
            pltpu.make_async_copy(x_hbm.at[pl.ds(0, CHUNK)],
                                  rows3.at[b], gsem).wait()
            pltpu.async_copy(rows3.at[b], acc_sh.at[eidx.at[ib, 1, j]],
                             ssem, add=True)

            @pl.when(t + 3 < t_total)
            def _():
                @pl.when(t >= 1)
                def _():
                    pltpu.make_async_copy(x_hbm.at[pl.ds(0, CHUNK)],
                                          rows3.at[0], ssem).wait()

                @pl.when(jnp.logical_and(j == 0, i + 2 < n_my))
                def _():
                    g = (i + 2) * NTILES + s
                    pltpu.async_copy(edge_hbm.at[:, g],
                                     eidx.at[lax.rem(i + 2, 3)], isem)

                tn = t + 3
                i2, j2 = lax.div(tn, GRP), lax.rem(tn, GRP)

                @pl.when(jnp.logical_and(j2 == 0, i2 >= 2))
                def _():
                    pltpu.make_async_copy(edge_hbm.at[:, s], eidx.at[0],
                                          isem).wait()

                pltpu.async_copy(x_hbm.at[eidx.at[lax.rem(i2, 3), 0, j2]],
                                 rows3.at[lax.rem(tn, 4)], gsem)

            return carry

        lax.fori_loop(0, t_total, _step, 0)
        for _ in range(4):
            pltpu.make_async_copy(x_hbm.at[pl.ds(0, CHUNK)],
                                  rows3.at[0], ssem).wait()

    @pl.when(r == 0)
    def _():
        _run(edge_sj_hbm)

    @pl.when(r == 1)
    def _():
        _run(edge_sr_hbm)

    plsc.subcore_barrier()

    pltpu.sync_copy(acc_sh.at[pl.ds(zstart, WRT)],
                    out_hbm.at[r, pl.ds(zstart, WRT)])

    @pl.when(s == NTILES - 1)
    def _():
        pltpu.sync_copy(acc_sh.at[pl.ds(NTILES * WRT, WTAIL)],
                        out_hbm.at[r, pl.ds(NTILES * WRT, WTAIL)])


_sc_agg = functools.partial(
    pl.kernel,
    out_type=jax.ShapeDtypeStruct((2, N, D), jnp.float32),
    mesh=plsc.VectorSubcoreMesh(core_axis_name="c", subcore_axis_name="s"),
    scratch_types=[
        pltpu.VMEM((3, 2, GRP, CHUNK), jnp.int32),
        pltpu.VMEM((4, CHUNK, D), jnp.float32),
        pltpu.VMEM_SHARED((N, D), jnp.float32),
        pltpu.SemaphoreType.DMA,
        pltpu.SemaphoreType.DMA,
        pltpu.SemaphoreType.DMA,
    ],
)(_sc_agg_body)


RB = 2000


def _tc_epilogue_body(aggj_ref, xj_ref, aggr_ref, xr_ref,
                      wrelj_ref, wrootj_ref, bj_ref,
                      wrelr_ref, wrootr_ref, br_ref,
                      wm1a_ref, wm1b_ref, bm1_ref,
                      wm2_ref, bm2_ref, wm3_ref, bm3_ref, out_ref):
    f32 = jnp.float32
    bf16 = jnp.bfloat16
    hj = (jnp.dot(aggj_ref[...], wrelj_ref[...], preferred_element_type=f32)
          + jnp.dot(xj_ref[...], wrootj_ref[...], preferred_element_type=f32)
          + bj_ref[...])
    hj = jnp.maximum(hj, 0.0)
    hr = (jnp.dot(aggr_ref[...], wrelr_ref[...], preferred_element_type=f32)
          + jnp.dot(xr_ref[...], wrootr_ref[...], preferred_element_type=f32)
          + br_ref[...])
    hr = jnp.maximum(hr, 0.0)
    h1 = (jnp.dot(hj.astype(bf16), wm1a_ref[...], preferred_element_type=f32)
          + jnp.dot(hr.astype(bf16), wm1b_ref[...], preferred_element_type=f32)
          + bm1_ref[...])
    h1 = jnp.maximum(h1, 0.0)
    h2 = jnp.maximum(
        jnp.dot(h1.astype(bf16), wm2_ref[...], preferred_element_type=f32)
        + bm2_ref[...],
        0.0)
    out_ref[...] = (jnp.sum(h2 * wm3_ref[...], axis=1, keepdims=True)
                    + bm3_ref[...])


def _full_spec(shape):
    return pl.BlockSpec(shape, lambda i: (0,) * len(shape))


def _row_spec():
    return pl.BlockSpec((RB, D), lambda i: (i, 0))


_tc_epilogue = pl.pallas_call(
    _tc_epilogue_body,
    grid=(N // RB,),
    in_specs=[
        _row_spec(), _row_spec(), _row_spec(), _row_spec(),
        _full_spec((D, D)), _full_spec((D, D)), _full_spec((1, D)),
        _full_spec((D, D)), _full_spec((D, D)), _full_spec((1, D)),
        _full_spec((D, H1)), _full_spec((D, H1)), _full_spec((1, H1)),
        _full_spec((H1, H2)), _full_spec((1, H2)),
        _full_spec((1, H2)), _full_spec((1, 1)),
    ],
    out_specs=pl.BlockSpec((RB, 1), lambda i: (i, 0)),
    out_shape=jax.ShapeDtypeStruct((N, 1), jnp.float32),
)


def kernel(x_skill, x_job, x_resume, edge_index_skill_job,
           edge_index_skill_resume, W_rel_sj, b_rel_sj, W_root_sj, W_rel_sr,
           b_rel_sr, W_root_sr, Wm1, bm1, Wm2, bm2, Wm3, bm3):
    edge_sj = edge_index_skill_job.reshape(2, NGRP, GRP, CHUNK)
    edge_sr = edge_index_skill_resume.reshape(2, NGRP, GRP, CHUNK)
    zeros = jnp.zeros((WRT, D), jnp.float32)
    agg = _sc_agg(x_skill, edge_sj, edge_sr, zeros)

    out = _tc_epilogue(
        agg[0], x_job, agg[1], x_resume,
        W_rel_sj.T, W_root_sj.T, b_rel_sj.reshape(1, D),
        W_rel_sr.T, W_root_sr.T, b_rel_sr.reshape(1, D),
        Wm1.T[:D].astype(jnp.bfloat16), Wm1.T[D:].astype(jnp.bfloat16),
        bm1.reshape(1, H1),
        Wm2.T.astype(jnp.bfloat16), bm2.reshape(1, H2),
        Wm3, bm3.reshape(1, 1),
    )
    return out.reshape(N)
